# baseline (speedup 1.0000x reference)
import functools

import jax
import jax.numpy as jnp
from jax import lax
from jax.experimental import pallas as pl
from jax.experimental.pallas import tpu as pltpu
from jax.experimental.pallas import tpu_sc as plsc

_BATCH = 16384
_FEAT = 64
_NC = 2
_NS = 16
_NW = _NC * _NS
_BPW = _BATCH // _NW
_CHUNK = 64
_NCHUNK = _BPW // _CHUNK
_LANES = 16
_NVR = _FEAT // _LANES

_mesh = plsc.VectorSubcoreMesh(core_axis_name="c", subcore_axis_name="s")


@functools.partial(
    pl.kernel,
    mesh=_mesh,
    out_type=jax.ShapeDtypeStruct((_NW * _LANES,), jnp.float32),
    scratch_types=[
        pltpu.VMEM((_NCHUNK, _CHUNK), jnp.int32),
        pltpu.VMEM((_BPW * _FEAT,), jnp.float32),
        pltpu.VMEM((_BPW * _FEAT,), jnp.float32),
        pltpu.VMEM((_LANES,), jnp.float32),
        pltpu.SemaphoreType.DMA,
        pltpu.SemaphoreType.DMA,
    ],
)
def _center_loss_sc(feat_hbm, lab_hbm, cent_hbm, out_hbm,
                    idx_v, cen_v, feat_v, acc_v, gsem, fsem):
    wid = lax.axis_index("s") * _NC + lax.axis_index("c")
    base = wid * _BPW

    fcopy = pltpu.async_copy(feat_hbm.at[pl.ds(base * _FEAT, _BPW * _FEAT)],
                             feat_v, fsem)

    for j in range(_NCHUNK):
        pltpu.sync_copy(lab_hbm.at[pl.ds(base + j * _CHUNK, _CHUNK)],
                        idx_v.at[j])

    def fetch(c):
        handles = []
        for v in range(_CHUNK // _LANES):
            rows = idx_v[c, pl.ds(v * _LANES, _LANES)]
            for k in range(_LANES):
                s = c * _CHUNK + v * _LANES + k
                handles.append(pltpu.async_copy(
                    cent_hbm.at[pl.ds(rows[k] * _FEAT, _FEAT)],
                    cen_v.at[pl.ds(s * _FEAT, _FEAT)],
                    gsem))
        return handles

    copies = [fetch(c) for c in range(_NCHUNK)]
    accs = tuple(jnp.zeros((_LANES,), jnp.float32) for _ in range(_NVR))
    for c in range(_NCHUNK):
        for h in copies[c]:
            h.wait()
        if c == 0:
            fcopy.wait()

        def body(s, accs, c=c):
            off = (c * _CHUNK + s) * _FEAT
            res = []
            for g in range(_NVR):
                d = (feat_v[pl.ds(off + g * _LANES, _LANES)]
                     - cen_v[pl.ds(off + g * _LANES, _LANES)])
                res.append(accs[g] + d * d)
            return tuple(res)

        accs = lax.fori_loop(0, _CHUNK, body, accs)

    acc = accs[0] + accs[1] + accs[2] + accs[3]
    acc_v[...] = acc * (1.0 / (2.0 * _BATCH))
    pltpu.sync_copy(acc_v, out_hbm.at[pl.ds(wid * _LANES, _LANES)])


def kernel(features, labels, centers):
    partials = _center_loss_sc(features.reshape(-1),
                               labels.astype(jnp.int32),
                               centers.reshape(-1))
    return jnp.sum(partials)


# speedup vs baseline: 1.0078x; 1.0078x over previous
import functools

import jax
import jax.numpy as jnp
from jax import lax
from jax.experimental import pallas as pl
from jax.experimental.pallas import tpu as pltpu
from jax.experimental.pallas import tpu_sc as plsc

_BATCH = 16384
_FEAT = 64
_NC = 2
_NS = 16
_NW = _NC * _NS
_BPW = _BATCH // _NW
_CHUNK = 64
_NCHUNK = _BPW // _CHUNK
_LANES = 16
_NVR = _FEAT // _LANES

_mesh = plsc.VectorSubcoreMesh(core_axis_name="c", subcore_axis_name="s")


@functools.partial(
    pl.kernel,
    mesh=_mesh,
    out_type=jax.ShapeDtypeStruct((_NW * _LANES,), jnp.float32),
    scratch_types=[
        pltpu.VMEM((_NCHUNK, _CHUNK), jnp.int32),
        pltpu.VMEM((_BPW * _FEAT,), jnp.float32),
        pltpu.VMEM((_BPW * _FEAT,), jnp.float32),
        pltpu.VMEM((_LANES,), jnp.float32),
        pltpu.SemaphoreType.DMA((_NCHUNK,)),
        pltpu.SemaphoreType.DMA,
    ],
)
def _center_loss_sc(feat_hbm, lab_hbm, cent_hbm, out_hbm,
                    idx_v, cen_v, feat_v, acc_v, gsems, fsem):
    wid = lax.axis_index("s") * _NC + lax.axis_index("c")
    base = wid * _BPW

    fcopy = pltpu.async_copy(feat_hbm.at[pl.ds(base * _FEAT, _BPW * _FEAT)],
                             feat_v, fsem)

    for j in range(_NCHUNK):
        pltpu.sync_copy(lab_hbm.at[pl.ds(base + j * _CHUNK, _CHUNK)],
                        idx_v.at[j])

    def fetch(c):
        for v in range(_CHUNK // _LANES):
            rows = idx_v[c, pl.ds(v * _LANES, _LANES)]
            for k in range(_LANES):
                s = c * _CHUNK + v * _LANES + k
                pltpu.async_copy(
                    cent_hbm.at[pl.ds(rows[k] * _FEAT, _FEAT)],
                    cen_v.at[pl.ds(s * _FEAT, _FEAT)],
                    gsems.at[c])

    for c in range(_NCHUNK):
        fetch(c)
    accs = tuple(jnp.zeros((_LANES,), jnp.float32) for _ in range(_NVR))
    for c in range(_NCHUNK):
        pltpu.make_async_copy(
            cent_hbm.at[pl.ds(0, _CHUNK * _FEAT)],
            cen_v.at[pl.ds(c * _CHUNK * _FEAT, _CHUNK * _FEAT)],
            gsems.at[c]).wait()
        if c == 0:
            fcopy.wait()

        def body(s, accs, c=c):
            off = (c * _CHUNK + s) * _FEAT
            res = []
            for g in range(_NVR):
                d = (feat_v[pl.ds(off + g * _LANES, _LANES)]
                     - cen_v[pl.ds(off + g * _LANES, _LANES)])
                res.append(accs[g] + d * d)
            return tuple(res)

        accs = lax.fori_loop(0, _CHUNK, body, accs)

    acc = accs[0] + accs[1] + accs[2] + accs[3]
    acc_v[...] = acc * (1.0 / (2.0 * _BATCH))
    pltpu.sync_copy(acc_v, out_hbm.at[pl.ds(wid * _LANES, _LANES)])


def kernel(features, labels, centers):
    partials = _center_loss_sc(features.reshape(-1),
                               labels.astype(jnp.int32),
                               centers.reshape(-1))
    return jnp.sum(partials)


# speedup vs baseline: 1.6846x; 1.6716x over previous
import functools

import jax
import jax.numpy as jnp
from jax import lax
from jax.experimental import pallas as pl
from jax.experimental.pallas import tpu as pltpu
from jax.experimental.pallas import tpu_sc as plsc

_BATCH = 16384
_FEAT = 64
_NC = 2
_NS = 16
_NW = _NC * _NS
_BPW = _BATCH // _NW
_CHUNK = 64
_NCHUNK = _BPW // _CHUNK
_LANES = 16
_NVR = _FEAT // _LANES

_mesh = plsc.VectorSubcoreMesh(core_axis_name="c", subcore_axis_name="s")


@functools.partial(
    pl.kernel,
    mesh=_mesh,
    out_type=jax.ShapeDtypeStruct((_NW * _LANES,), jnp.float32),
    scratch_types=[
        pltpu.VMEM((_NCHUNK, _CHUNK), jnp.int32),
        pltpu.VMEM((2, _CHUNK, _FEAT), jnp.float32),
        pltpu.VMEM((_BPW, _FEAT), jnp.float32),
        pltpu.VMEM((_LANES,), jnp.float32),
        pltpu.SemaphoreType.DMA,
        pltpu.SemaphoreType.DMA,
        pltpu.SemaphoreType.DMA,
    ],
)
def _center_loss_sc(feat_hbm, lab_hbm, cent_hbm, out_hbm,
                    idx_v, cen_v, feat_v, acc_v, gsem0, gsem1, fsem):
    wid = lax.axis_index("s") * _NC + lax.axis_index("c")
    base = wid * _BPW
    gsems = (gsem0, gsem1)

    fcopy = pltpu.async_copy(feat_hbm.at[pl.ds(base, _BPW)], feat_v, fsem)

    for j in range(_NCHUNK):
        pltpu.sync_copy(lab_hbm.at[pl.ds(base + j * _CHUNK, _CHUNK)],
                        idx_v.at[j])

    def fetch(c):
        handles = []
        for v in range(_CHUNK // _LANES):
            rows = idx_v[c, pl.ds(v * _LANES, _LANES)]
            for k in range(_LANES):
                s = v * _LANES + k
                handles.append(pltpu.async_copy(
                    cent_hbm.at[pl.ds(rows[k], 1), :],
                    cen_v.at[c % 2, pl.ds(s, 1), :],
                    gsems[c % 2]))
        return handles

    copies = [fetch(0)]
    accs = tuple(jnp.zeros((_LANES,), jnp.float32) for _ in range(_NVR))
    for c in range(_NCHUNK):
        if c + 1 < _NCHUNK:
            copies.append(fetch(c + 1))
        for h in copies[c]:
            h.wait()
        if c == 0:
            fcopy.wait()

        def body(s, accs, c=c):
            res = []
            for g in range(_NVR):
                d = (feat_v[c * _CHUNK + s, pl.ds(g * _LANES, _LANES)]
                     - cen_v[c % 2, s, pl.ds(g * _LANES, _LANES)])
                res.append(accs[g] + d * d)
            return tuple(res)

        accs = lax.fori_loop(0, _CHUNK, body, accs)

    acc = accs[0] + accs[1] + accs[2] + accs[3]
    acc_v[...] = acc * (1.0 / (2.0 * _BATCH))
    pltpu.sync_copy(acc_v, out_hbm.at[pl.ds(wid * _LANES, _LANES)])


def kernel(features, labels, centers):
    partials = _center_loss_sc(features, labels.astype(jnp.int32), centers)
    return jnp.sum(partials)
